# baseline (speedup 1.0000x reference)
import functools

import jax
import jax.numpy as jnp
from jax import lax
from jax.experimental import pallas as pl
from jax.experimental.pallas import tpu as pltpu
from jax.experimental.pallas import tpu_sc as plsc

_NC = 2
_NS = 16
_NW = _NC * _NS

_IDX_PER_DMA = 128
_DMAS_PER_CHUNK = 8
_CHUNK = _IDX_PER_DMA * _DMAS_PER_CHUNK

_PROJ_BLK = 8192
_LPAD = 64


def _proj_body(t0, t1, t2, t3, w4_ref, b4_ref, o_ref):
    d = t0.shape[0]
    acc = b4_ref[...]
    for k, tk in enumerate((t0, t1, t2, t3)):
        acc = acc + lax.dot_general(
            tk[...], w4_ref[pl.ds(k * d, d), :], (((0,), (0,)), ((), ())),
            preferred_element_type=jnp.float32,
        )
    o_ref[...] = acc


def _project_packed(tableT, W4, b4):
    d, rows = tableT.shape
    blk4 = _PROJ_BLK // 4
    g4 = (-(-rows // blk4) + 3) // 4
    max_blk = (rows - 1) // blk4
    tspec = lambda off: pl.BlockSpec(
        (d, blk4), lambda i, o=off: (0, jnp.minimum(i + o, max_blk)))
    return pl.pallas_call(
        _proj_body,
        grid=(g4,),
        in_specs=[
            tspec(0), tspec(g4), tspec(2 * g4), tspec(3 * g4),
            pl.BlockSpec((4 * d, 128), lambda i: (0, 0)),
            pl.BlockSpec((1, 128), lambda i: (0, 0)),
        ],
        out_specs=pl.BlockSpec((blk4, 128), lambda i: (i, 0)),
        out_shape=jax.ShapeDtypeStruct((g4 * blk4, 128), jnp.float32),
    )(tableT, tableT, tableT, tableT, W4, b4)


def _tr_body(nvalid, i_ref, o_ref):
    o_ref[...] = i_ref[...].T[:nvalid, :]


def _transpose_sliced(padded2d, nvalid, blk=512):
    n, m = padded2d.shape
    grid = n // blk
    return pl.pallas_call(
        functools.partial(_tr_body, nvalid),
        grid=(grid,),
        in_specs=[pl.BlockSpec((blk, m), lambda i: (i, 0))],
        out_specs=pl.BlockSpec((nvalid, blk), lambda i: (0, i)),
        out_shape=jax.ShapeDtypeStruct((nvalid, n), jnp.float32),
    )(padded2d)


def _make_gather(n_tokens, nf):
    assert n_tokens % (_NW * _CHUNK) == 0
    per_w = n_tokens // _NW
    chunks = per_w // _CHUNK
    idx_rows = per_w // _IDX_PER_DMA

    mesh = plsc.VectorSubcoreMesh(
        core_axis_name="c", subcore_axis_name="s",
        num_cores=_NC, num_subcores=_NS,
    )

    @functools.partial(
        pl.kernel,
        out_type=jax.ShapeDtypeStruct((n_tokens, nf), jnp.float32),
        mesh=mesh,
        compiler_params=pltpu.CompilerParams(use_tc_tiling_on_sc=False),
        scratch_types=[
            pltpu.VMEM((idx_rows, _IDX_PER_DMA), jnp.int32),
            pltpu.VMEM((_CHUNK, nf), jnp.float32),
            pltpu.SemaphoreType.DMA,
        ],
    )
    def gather(tw_hbm, idx_hbm, out_hbm, idx_v, rows_v, sem):
        wid = lax.axis_index("s") * _NC + lax.axis_index("c")
        row0 = wid * idx_rows
        pltpu.sync_copy(idx_hbm.at[pl.ds(row0, idx_rows)], idx_v)

        def chunk_body(g, carry):
            copies = [
                pltpu.async_copy(
                    tw_hbm.at[idx_v.at[g * _DMAS_PER_CHUNK + j]],
                    rows_v.at[pl.ds(j * _IDX_PER_DMA, _IDX_PER_DMA)],
                    sem,
                )
                for j in range(_DMAS_PER_CHUNK)
            ]
            for cp in copies:
                cp.wait()
            pltpu.sync_copy(
                rows_v,
                out_hbm.at[pl.ds(wid * per_w + g * _CHUNK, _CHUNK)],
            )
            return carry

        lax.fori_loop(0, chunks, chunk_body, 0)

    return gather


def kernel(x, table, W, b):
    B, L = x.shape
    nf, d = W.shape
    npack = 128 // nf
    W4 = jax.scipy.linalg.block_diag(*([W.T] * npack))
    b4 = jnp.tile(b, npack).reshape(1, 128)
    tw_packed = _project_packed(table.T, W4, b4)
    r4 = tw_packed.shape[0]
    tw32 = tw_packed.reshape(r4 * npack, nf)
    v = jnp.pad(x + 1, ((0, 0), (0, _LPAD - L)))
    sub = npack * (v % r4) + v // r4
    idx = sub.reshape(-1, _IDX_PER_DMA)
    flat = _make_gather(B * _LPAD, nf)(tw32, idx)
    out2d = _transpose_sliced(flat.reshape(B, _LPAD * nf), L * nf)
    return jnp.transpose(out2d.reshape(L, nf, B), (2, 0, 1))


# speedup vs baseline: 5.0865x; 5.0865x over previous
import functools

import jax
import jax.numpy as jnp
from jax import lax
from jax.experimental import pallas as pl
from jax.experimental.pallas import tpu as pltpu
from jax.experimental.pallas import tpu_sc as plsc

_NC = 2
_NS = 16
_NW = _NC * _NS

_IDX_PER_DMA = 128
_DMAS_PER_CHUNK = 8
_CHUNK = _IDX_PER_DMA * _DMAS_PER_CHUNK

_PROJ_BLK = 8192
_LPAD = 64


def _proj_body(t0, t1, t2, t3, w4_ref, b4_ref, o_ref):
    d = t0.shape[0]
    acc = b4_ref[...]
    for k, tk in enumerate((t0, t1, t2, t3)):
        acc = acc + lax.dot_general(
            tk[...], w4_ref[pl.ds(k * d, d), :], (((0,), (0,)), ((), ())),
            preferred_element_type=jnp.float32,
        )
    o_ref[...] = acc


def _project_packed(tableT, W4, b4):
    d, rows = tableT.shape
    blk4 = _PROJ_BLK // 4
    g4 = (-(-rows // blk4) + 3) // 4
    max_blk = (rows - 1) // blk4
    tspec = lambda off: pl.BlockSpec(
        (d, blk4), lambda i, o=off: (0, jnp.minimum(i + o, max_blk)))
    return pl.pallas_call(
        _proj_body,
        grid=(g4,),
        in_specs=[
            tspec(0), tspec(g4), tspec(2 * g4), tspec(3 * g4),
            pl.BlockSpec((4 * d, 128), lambda i: (0, 0)),
            pl.BlockSpec((1, 128), lambda i: (0, 0)),
        ],
        out_specs=pl.BlockSpec((blk4, 128), lambda i: (i, 0)),
        out_shape=jax.ShapeDtypeStruct((g4 * blk4, 128), jnp.float32),
    )(tableT, tableT, tableT, tableT, W4, b4)


def _tr_body(nvalid, i_ref, o_ref):
    o_ref[...] = i_ref[...].T[:nvalid, :]


def _transpose_sliced(padded2d, nvalid, blk=512):
    n, m = padded2d.shape
    grid = n // blk
    return pl.pallas_call(
        functools.partial(_tr_body, nvalid),
        grid=(grid,),
        in_specs=[pl.BlockSpec((blk, m), lambda i: (i, 0))],
        out_specs=pl.BlockSpec((nvalid, blk), lambda i: (0, i)),
        out_shape=jax.ShapeDtypeStruct((nvalid, n), jnp.float32),
    )(padded2d)


def _make_gather(n_tokens, nf):
    assert n_tokens % (_NW * _CHUNK) == 0
    per_w = n_tokens // _NW
    chunks = per_w // _CHUNK
    idx_rows = per_w // _IDX_PER_DMA

    mesh = plsc.VectorSubcoreMesh(
        core_axis_name="c", subcore_axis_name="s",
        num_cores=_NC, num_subcores=_NS,
    )

    @functools.partial(
        pl.kernel,
        out_type=jax.ShapeDtypeStruct((n_tokens, nf), jnp.float32),
        mesh=mesh,
        compiler_params=pltpu.CompilerParams(use_tc_tiling_on_sc=False),
        scratch_types=[
            pltpu.VMEM((idx_rows, _IDX_PER_DMA), jnp.int32),
            pltpu.VMEM((_CHUNK, nf), jnp.float32),
            pltpu.SemaphoreType.DMA,
        ],
    )
    def gather(tw_hbm, idx_hbm, out_hbm, idx_v, rows_v, sem):
        wid = lax.axis_index("s") * _NC + lax.axis_index("c")
        row0 = wid * idx_rows
        pltpu.sync_copy(idx_hbm.at[pl.ds(row0, idx_rows)], idx_v)

        def chunk_body(g, carry):
            copies = [
                pltpu.async_copy(
                    tw_hbm.at[idx_v.at[g * _DMAS_PER_CHUNK + j]],
                    rows_v.at[pl.ds(j * _IDX_PER_DMA, _IDX_PER_DMA)],
                    sem,
                )
                for j in range(_DMAS_PER_CHUNK)
            ]
            for cp in copies:
                cp.wait()
            pltpu.sync_copy(
                rows_v,
                out_hbm.at[pl.ds(wid * per_w + g * _CHUNK, _CHUNK)],
            )
            return carry

        lax.fori_loop(0, chunks, chunk_body, 0)

    return gather


def kernel(x, table, W, b):
    B, L = x.shape
    nf, d = W.shape
    npack = 128 // nf
    W4 = jax.scipy.linalg.block_diag(*([W.T] * npack))
    b4 = jnp.tile(b, npack).reshape(1, 128)
    tw_packed = _project_packed(table.T, W4, b4)
    r4 = tw_packed.shape[0]
    tw32 = tw_packed.reshape(r4 * npack, nf)
    sub = npack * ((x + 1) % r4) + (x + 1) // r4
    npad = B * (_LPAD - L)
    pad_sub = (jnp.arange(npad, dtype=jnp.int32) % (npack * r4)).reshape(
        B, _LPAD - L)
    sub = jnp.concatenate([sub, pad_sub], axis=1)
    idx = sub.reshape(-1, _IDX_PER_DMA)
    flat = _make_gather(B * _LPAD, nf)(tw32, idx)
    out2d = _transpose_sliced(flat.reshape(B, _LPAD * nf), L * nf)
    return jnp.transpose(out2d.reshape(L, nf, B), (2, 0, 1))


# speedup vs baseline: 5.7253x; 1.1256x over previous
import functools

import jax
import jax.numpy as jnp
from jax import lax
from jax.experimental import pallas as pl
from jax.experimental.pallas import tpu as pltpu
from jax.experimental.pallas import tpu_sc as plsc

_NC = 2
_NS = 16
_NW = _NC * _NS

_IDX_PER_DMA = 128
_DMAS_PER_CHUNK = 8
_CHUNK = _IDX_PER_DMA * _DMAS_PER_CHUNK

_PROJ_BLK = 16384


def _proj_body(t0, t1, t2, t3, w4_ref, b4_ref, o_ref):
    d = t0.shape[0]
    acc = b4_ref[...]
    for k, tk in enumerate((t0, t1, t2, t3)):
        acc = acc + lax.dot_general(
            tk[...], w4_ref[pl.ds(k * d, d), :], (((0,), (0,)), ((), ())),
            preferred_element_type=jnp.float32,
        )
    o_ref[...] = acc


def _project_packed(tableT, W4, b4):
    d, rows = tableT.shape
    blk4 = _PROJ_BLK // 4
    g4 = (-(-rows // blk4) + 3) // 4
    max_blk = (rows - 1) // blk4
    tspec = lambda off: pl.BlockSpec(
        (d, blk4), lambda i, o=off: (0, jnp.minimum(i + o, max_blk)))
    return pl.pallas_call(
        _proj_body,
        grid=(g4,),
        in_specs=[
            tspec(0), tspec(g4), tspec(2 * g4), tspec(3 * g4),
            pl.BlockSpec((4 * d, 128), lambda i: (0, 0)),
            pl.BlockSpec((1, 128), lambda i: (0, 0)),
        ],
        out_specs=pl.BlockSpec((blk4, 128), lambda i: (i, 0)),
        out_shape=jax.ShapeDtypeStruct((g4 * blk4, 128), jnp.float32),
    )(tableT, tableT, tableT, tableT, W4, b4)


def _tr_body(i_ref, o_ref):
    o_ref[...] = i_ref[...].T


def _transpose(flat2d, blk=1024):
    n, m = flat2d.shape
    grid = n // blk
    return pl.pallas_call(
        _tr_body,
        grid=(grid,),
        in_specs=[pl.BlockSpec((blk, m), lambda i: (i, 0))],
        out_specs=pl.BlockSpec((m, blk), lambda i: (0, i)),
        out_shape=jax.ShapeDtypeStruct((m, n), jnp.float32),
    )(flat2d)


def _make_gather(n_tokens, nf):
    assert n_tokens % (_NW * _CHUNK) == 0
    per_w = n_tokens // _NW
    chunks = per_w // _CHUNK
    idx_rows = per_w // _IDX_PER_DMA

    mesh = plsc.VectorSubcoreMesh(
        core_axis_name="c", subcore_axis_name="s",
        num_cores=_NC, num_subcores=_NS,
    )

    @functools.partial(
        pl.kernel,
        out_type=jax.ShapeDtypeStruct((n_tokens, nf), jnp.float32),
        mesh=mesh,
        compiler_params=pltpu.CompilerParams(use_tc_tiling_on_sc=False),
        scratch_types=[
            pltpu.VMEM((idx_rows, _IDX_PER_DMA), jnp.int32),
            pltpu.VMEM((_CHUNK, nf), jnp.float32),
            pltpu.SemaphoreType.DMA,
        ],
    )
    def gather(tw_hbm, idx_hbm, out_hbm, idx_v, rows_v, sem):
        wid = lax.axis_index("s") * _NC + lax.axis_index("c")
        row0 = wid * idx_rows
        pltpu.sync_copy(idx_hbm.at[pl.ds(row0, idx_rows)], idx_v)

        def chunk_body(g, carry):
            copies = [
                pltpu.async_copy(
                    tw_hbm.at[idx_v.at[g * _DMAS_PER_CHUNK + j]],
                    rows_v.at[pl.ds(j * _IDX_PER_DMA, _IDX_PER_DMA)],
                    sem,
                )
                for j in range(_DMAS_PER_CHUNK)
            ]
            for cp in copies:
                cp.wait()
            pltpu.sync_copy(
                rows_v,
                out_hbm.at[pl.ds(wid * per_w + g * _CHUNK, _CHUNK)],
            )
            return carry

        lax.fori_loop(0, chunks, chunk_body, 0)

    return gather


def kernel(x, table, W, b):
    B, L = x.shape
    nf, d = W.shape
    npack = 128 // nf
    W4 = jax.scipy.linalg.block_diag(*([W.T] * npack))
    b4 = jnp.tile(b, npack).reshape(1, 128)
    tw_packed = _project_packed(table.T, W4, b4)
    tw32 = tw_packed.reshape(tw_packed.shape[0] * npack, nf)
    r4 = tw_packed.shape[0]
    v = x.reshape(-1) + 1
    sub = npack * (v % r4) + v // r4
    idx = sub.reshape(-1, _IDX_PER_DMA)
    flat = _make_gather(B * L, nf)(tw32, idx)
    out2d = _transpose(flat.reshape(B, L * nf))
    return jnp.transpose(out2d.reshape(L, nf, B), (2, 0, 1))


# speedup vs baseline: 5.9514x; 1.0395x over previous
import functools

import jax
import jax.numpy as jnp
from jax import lax
from jax.experimental import pallas as pl
from jax.experimental.pallas import tpu as pltpu
from jax.experimental.pallas import tpu_sc as plsc

_NC = 2
_NS = 16
_NW = _NC * _NS

_IDX_PER_DMA = 128
_DMAS_PER_CHUNK = 8
_CHUNK = _IDX_PER_DMA * _DMAS_PER_CHUNK

_PROJ_BLK = 32768


def _proj_body(t0, t1, t2, t3, w4_ref, b4_ref, o_ref):
    d = t0.shape[0]
    acc = b4_ref[...]
    for k, tk in enumerate((t0, t1, t2, t3)):
        acc = acc + lax.dot_general(
            tk[...], w4_ref[pl.ds(k * d, d), :], (((0,), (0,)), ((), ())),
            preferred_element_type=jnp.float32,
        )
    o_ref[...] = acc


def _project_packed(tableT, W4, b4):
    d, rows = tableT.shape
    blk4 = _PROJ_BLK // 4
    g4 = (-(-rows // blk4) + 3) // 4
    max_blk = (rows - 1) // blk4
    tspec = lambda off: pl.BlockSpec(
        (d, blk4), lambda i, o=off: (0, jnp.minimum(i + o, max_blk)))
    return pl.pallas_call(
        _proj_body,
        grid=(g4,),
        in_specs=[
            tspec(0), tspec(g4), tspec(2 * g4), tspec(3 * g4),
            pl.BlockSpec((4 * d, 128), lambda i: (0, 0)),
            pl.BlockSpec((1, 128), lambda i: (0, 0)),
        ],
        out_specs=pl.BlockSpec((blk4, 128), lambda i: (i, 0)),
        out_shape=jax.ShapeDtypeStruct((g4 * blk4, 128), jnp.float32),
    )(tableT, tableT, tableT, tableT, W4, b4)


def _tr_body(i_ref, o_ref):
    o_ref[...] = i_ref[...].T


def _transpose(flat2d, blk=1024):
    n, m = flat2d.shape
    grid = n // blk
    return pl.pallas_call(
        _tr_body,
        grid=(grid,),
        in_specs=[pl.BlockSpec((blk, m), lambda i: (i, 0))],
        out_specs=pl.BlockSpec((m, blk), lambda i: (0, i)),
        out_shape=jax.ShapeDtypeStruct((m, n), jnp.float32),
    )(flat2d)


def _make_gather(n_tokens, nf):
    assert n_tokens % (_NW * _CHUNK) == 0
    per_w = n_tokens // _NW
    chunks = per_w // _CHUNK
    idx_rows = per_w // _IDX_PER_DMA

    mesh = plsc.VectorSubcoreMesh(
        core_axis_name="c", subcore_axis_name="s",
        num_cores=_NC, num_subcores=_NS,
    )

    @functools.partial(
        pl.kernel,
        out_type=jax.ShapeDtypeStruct((n_tokens, nf), jnp.float32),
        mesh=mesh,
        compiler_params=pltpu.CompilerParams(use_tc_tiling_on_sc=False),
        scratch_types=[
            pltpu.VMEM((idx_rows, _IDX_PER_DMA), jnp.int32),
            pltpu.VMEM((_CHUNK, nf), jnp.float32),
            pltpu.SemaphoreType.DMA,
        ],
    )
    def gather(tw_hbm, idx_hbm, out_hbm, idx_v, rows_v, sem):
        wid = lax.axis_index("s") * _NC + lax.axis_index("c")
        row0 = wid * idx_rows
        pltpu.sync_copy(idx_hbm.at[pl.ds(row0, idx_rows)], idx_v)

        def chunk_body(g, carry):
            copies = [
                pltpu.async_copy(
                    tw_hbm.at[idx_v.at[g * _DMAS_PER_CHUNK + j]],
                    rows_v.at[pl.ds(j * _IDX_PER_DMA, _IDX_PER_DMA)],
                    sem,
                )
                for j in range(_DMAS_PER_CHUNK)
            ]
            for cp in copies:
                cp.wait()
            pltpu.sync_copy(
                rows_v,
                out_hbm.at[pl.ds(wid * per_w + g * _CHUNK, _CHUNK)],
            )
            return carry

        lax.fori_loop(0, chunks, chunk_body, 0)

    return gather


def kernel(x, table, W, b):
    B, L = x.shape
    nf, d = W.shape
    npack = 128 // nf
    W4 = jax.scipy.linalg.block_diag(*([W.T] * npack))
    b4 = jnp.tile(b, npack).reshape(1, 128)
    tw_packed = _project_packed(table.T, W4, b4)
    tw32 = tw_packed.reshape(tw_packed.shape[0] * npack, nf)
    r4 = tw_packed.shape[0]
    v = x.reshape(-1) + 1
    sub = npack * (v % r4) + v // r4
    idx = sub.reshape(-1, _IDX_PER_DMA)
    flat = _make_gather(B * L, nf)(tw32, idx)
    out2d = _transpose(flat.reshape(B, L * nf))
    return jnp.transpose(out2d.reshape(L, nf, B), (2, 0, 1))


# speedup vs baseline: 6.4163x; 1.0781x over previous
import functools

import jax
import jax.numpy as jnp
from jax import lax
from jax.experimental import pallas as pl
from jax.experimental.pallas import tpu as pltpu
from jax.experimental.pallas import tpu_sc as plsc

_NC = 2
_NS = 16
_NW = _NC * _NS

_IDX_PER_DMA = 128
_DMAS_PER_CHUNK = 8
_CHUNK = _IDX_PER_DMA * _DMAS_PER_CHUNK

_PROJ_BLK = 65536


def _proj_body(t0, t1, t2, t3, w4_ref, b4_ref, o_ref):
    d = t0.shape[0]
    acc = b4_ref[...]
    for k, tk in enumerate((t0, t1, t2, t3)):
        acc = acc + lax.dot_general(
            tk[...], w4_ref[pl.ds(k * d, d), :], (((0,), (0,)), ((), ())),
            preferred_element_type=jnp.float32,
        )
    o_ref[...] = acc


def _project_packed(tableT, W4, b4):
    d, rows = tableT.shape
    blk4 = _PROJ_BLK // 4
    g4 = (-(-rows // blk4) + 3) // 4
    max_blk = (rows - 1) // blk4
    tspec = lambda off: pl.BlockSpec(
        (d, blk4), lambda i, o=off: (0, jnp.minimum(i + o, max_blk)))
    return pl.pallas_call(
        _proj_body,
        grid=(g4,),
        in_specs=[
            tspec(0), tspec(g4), tspec(2 * g4), tspec(3 * g4),
            pl.BlockSpec((4 * d, 128), lambda i: (0, 0)),
            pl.BlockSpec((1, 128), lambda i: (0, 0)),
        ],
        out_specs=pl.BlockSpec((blk4, 128), lambda i: (i, 0)),
        out_shape=jax.ShapeDtypeStruct((g4 * blk4, 128), jnp.float32),
    )(tableT, tableT, tableT, tableT, W4, b4)


def _tr_body(i_ref, o_ref):
    o_ref[...] = i_ref[...].T


def _transpose(flat2d, blk=2048):
    n, m = flat2d.shape
    grid = n // blk
    return pl.pallas_call(
        _tr_body,
        grid=(grid,),
        in_specs=[pl.BlockSpec((blk, m), lambda i: (i, 0))],
        out_specs=pl.BlockSpec((m, blk), lambda i: (0, i)),
        out_shape=jax.ShapeDtypeStruct((m, n), jnp.float32),
    )(flat2d)


def _make_gather(n_tokens, nf):
    assert n_tokens % (_NW * _CHUNK) == 0
    per_w = n_tokens // _NW
    chunks = per_w // _CHUNK
    idx_rows = per_w // _IDX_PER_DMA

    mesh = plsc.VectorSubcoreMesh(
        core_axis_name="c", subcore_axis_name="s",
        num_cores=_NC, num_subcores=_NS,
    )

    @functools.partial(
        pl.kernel,
        out_type=jax.ShapeDtypeStruct((n_tokens, nf), jnp.float32),
        mesh=mesh,
        compiler_params=pltpu.CompilerParams(use_tc_tiling_on_sc=False),
        scratch_types=[
            pltpu.VMEM((idx_rows, _IDX_PER_DMA), jnp.int32),
            pltpu.VMEM((_CHUNK, nf), jnp.float32),
            pltpu.SemaphoreType.DMA,
        ],
    )
    def gather(tw_hbm, idx_hbm, out_hbm, idx_v, rows_v, sem):
        wid = lax.axis_index("s") * _NC + lax.axis_index("c")
        row0 = wid * idx_rows
        pltpu.sync_copy(idx_hbm.at[pl.ds(row0, idx_rows)], idx_v)

        def chunk_body(g, carry):
            copies = [
                pltpu.async_copy(
                    tw_hbm.at[idx_v.at[g * _DMAS_PER_CHUNK + j]],
                    rows_v.at[pl.ds(j * _IDX_PER_DMA, _IDX_PER_DMA)],
                    sem,
                )
                for j in range(_DMAS_PER_CHUNK)
            ]
            for cp in copies:
                cp.wait()
            pltpu.sync_copy(
                rows_v,
                out_hbm.at[pl.ds(wid * per_w + g * _CHUNK, _CHUNK)],
            )
            return carry

        lax.fori_loop(0, chunks, chunk_body, 0)

    return gather


def kernel(x, table, W, b):
    B, L = x.shape
    nf, d = W.shape
    npack = 128 // nf
    W4 = jax.scipy.linalg.block_diag(*([W.T] * npack))
    b4 = jnp.tile(b, npack).reshape(1, 128)
    tw_packed = _project_packed(table.T, W4, b4)
    tw32 = tw_packed.reshape(tw_packed.shape[0] * npack, nf)
    r4 = tw_packed.shape[0]
    v = x.reshape(-1) + 1
    sub = npack * (v % r4) + v // r4
    idx = sub.reshape(-1, _IDX_PER_DMA)
    flat = _make_gather(B * L, nf)(tw32, idx)
    out2d = _transpose(flat.reshape(B, L * nf))
    return jnp.transpose(out2d.reshape(L, nf, B), (2, 0, 1))
